# Initial kernel scaffold; baseline (speedup 1.0000x reference)
import jax
import jax.numpy as jnp
from jax.experimental import pallas as pl


def kernel(x, edge_index, hist0, agg_hist0, hist1, agg_hist1, W0, b0, W1, b1):
    raise NotImplementedError("write your pallas kernel here")


# speedup vs baseline: 6.4736x; 6.4736x over previous
import functools

import jax
import jax.numpy as jnp
from jax import lax
from jax.experimental import pallas as pl
from jax.experimental.pallas import tpu as pltpu
from jax.experimental.pallas import tpu_sc as plsc

N = 10000
E = 320000
D = 128
NC = 2
NS = 16
NW = NC * NS
EPW = E // NW
K = 80
CH = EPW // K
G = 5
CHG = CH // G
RPS = 624
TAIL = N - NS * RPS
TBASE = NS * RPS


def _sc_segsum_body(do_deg, *refs):
    if do_deg:
        (hdelta, src3d, dst3d, zfeat, msum_out, deg_out,
         acc, src_idx, dst_idx, rows, dloc, sem) = refs
    else:
        (hdelta, src3d, dst3d, zfeat, msum_out,
         acc, src_idx, dst_idx, rows, sem) = refs

    c = lax.axis_index("c")
    s = lax.axis_index("s")
    w = c * NS + s

    base = s * RPS
    pltpu.sync_copy(zfeat.at[pl.ds(base, RPS)], acc.at[pl.ds(base, RPS)])

    @pl.when(s == NS - 1)
    def _zero_tail():
        pltpu.sync_copy(zfeat.at[pl.ds(TBASE, TAIL)],
                        acc.at[pl.ds(TBASE, TAIL)])

    if do_deg:
        def zero_deg(i, _):
            dloc[pl.ds(i * 16, 16)] = jnp.zeros((16,), jnp.float32)
            return 0
        lax.fori_loop(0, N // 16, zero_deg, 0)
    del base

    plsc.subcore_barrier()

    def group(g, _):
        pltpu.sync_copy(src3d.at[w, g], src_idx)
        pltpu.sync_copy(dst3d.at[w, g], dst_idx)

        def chunk(j, _):
            pltpu.async_copy(hdelta.at[src_idx.at[j]], rows, sem).wait()
            pltpu.sync_copy(rows, acc.at[dst_idx.at[j]], add=True)
            if do_deg:
                def vec(l, _):
                    iv = dst_idx[j, pl.ds(l * 16, 16)]
                    plsc.addupdate_scatter(
                        dloc, [iv], jnp.ones((16,), jnp.float32))
                    return 0
                lax.fori_loop(0, K // 16, vec, 0)
            return 0

        lax.fori_loop(0, CHG, chunk, 0)
        return 0

    lax.fori_loop(0, G, group, 0)
    plsc.subcore_barrier()

    pltpu.sync_copy(acc.at[pl.ds(s * RPS, RPS)],
                    msum_out.at[c, pl.ds(s * RPS, RPS)])
    if do_deg:
        pltpu.sync_copy(dloc, deg_out.at[w])

    @pl.when(s == NS - 1)
    def _pub_tail():
        pltpu.sync_copy(acc.at[pl.ds(TBASE, TAIL)],
                        msum_out.at[c, pl.ds(TBASE, TAIL)])


@functools.cache
def _sc_kernels():
    mesh = plsc.VectorSubcoreMesh(core_axis_name="c", subcore_axis_name="s")
    segsum_deg = pl.kernel(
        functools.partial(_sc_segsum_body, True),
        out_type=[
            jax.ShapeDtypeStruct((NC, N, D), jnp.float32),
            jax.ShapeDtypeStruct((NW, N), jnp.float32),
        ],
        mesh=mesh,
        compiler_params=pltpu.CompilerParams(needs_layout_passes=False),
        scratch_types=[
            pltpu.VMEM_SHARED((N, D), jnp.float32),
            pltpu.VMEM((CHG, K), jnp.int32),
            pltpu.VMEM((CHG, K), jnp.int32),
            pltpu.VMEM((K, D), jnp.float32),
            pltpu.VMEM((N,), jnp.float32),
            pltpu.SemaphoreType.DMA,
        ],
    )
    segsum = pl.kernel(
        functools.partial(_sc_segsum_body, False),
        out_type=[jax.ShapeDtypeStruct((NC, N, D), jnp.float32)],
        mesh=mesh,
        scratch_types=[
            pltpu.VMEM_SHARED((N, D), jnp.float32),
            pltpu.VMEM((CHG, K), jnp.int32),
            pltpu.VMEM((CHG, K), jnp.int32),
            pltpu.VMEM((K, D), jnp.float32),
            pltpu.SemaphoreType.DMA,
        ],
    )
    return segsum_deg, segsum


_BN = 400
_GRID = N // _BN


def _tc_sub_body(x_ref, h_ref, o_ref):
    o_ref[...] = x_ref[...] - h_ref[...]


def _tc_sub(x, h):
    return pl.pallas_call(
        _tc_sub_body,
        grid=(_GRID,),
        in_specs=[
            pl.BlockSpec((_BN, D), lambda i: (i, 0)),
            pl.BlockSpec((_BN, D), lambda i: (i, 0)),
        ],
        out_specs=pl.BlockSpec((_BN, D), lambda i: (i, 0)),
        out_shape=jax.ShapeDtypeStruct((N, D), jnp.float32),
    )(x, h)


def _tc_layer0_body(x_ref, agg_ref, mp_ref, dp_ref, wa_ref, wb_ref, b_ref,
                    hist1_ref, h_ref, hd1_ref):
    mp = mp_ref[...]
    dp = dp_ref[...]
    deg = jnp.maximum(jnp.sum(dp, axis=1), 1.0)
    hn = agg_ref[...] + (mp[0] + mp[1]) * (1.0 / deg)[:, None]
    h = jnp.dot(x_ref[...], wa_ref[...], preferred_element_type=jnp.float32)
    h += jnp.dot(hn, wb_ref[...], preferred_element_type=jnp.float32)
    h = jnp.maximum(h + b_ref[...][None, :], 0.0)
    h_ref[...] = h
    hd1_ref[...] = h - hist1_ref[...]


def _tc_layer0(x, agg0, mp, dp, waT, wbT, b, hist1):
    return pl.pallas_call(
        _tc_layer0_body,
        grid=(_GRID,),
        in_specs=[
            pl.BlockSpec((_BN, D), lambda i: (i, 0)),
            pl.BlockSpec((_BN, D), lambda i: (i, 0)),
            pl.BlockSpec((NC, _BN, D), lambda i: (0, i, 0)),
            pl.BlockSpec((_BN, NW), lambda i: (i, 0)),
            pl.BlockSpec((D, D), lambda i: (0, 0)),
            pl.BlockSpec((D, D), lambda i: (0, 0)),
            pl.BlockSpec((D,), lambda i: (0,)),
            pl.BlockSpec((_BN, D), lambda i: (i, 0)),
        ],
        out_specs=[
            pl.BlockSpec((_BN, D), lambda i: (i, 0)),
            pl.BlockSpec((_BN, D), lambda i: (i, 0)),
        ],
        out_shape=[
            jax.ShapeDtypeStruct((N, D), jnp.float32),
            jax.ShapeDtypeStruct((N, D), jnp.float32),
        ],
    )(x, agg0, mp, dp, waT, wbT, b, hist1)


def _tc_layer1_body(h_ref, agg_ref, mp_ref, dp_ref, wa_ref, wb_ref, b_ref,
                    o_ref):
    mp = mp_ref[...]
    dp = dp_ref[...]
    deg = jnp.maximum(jnp.sum(dp, axis=1), 1.0)
    hn = agg_ref[...] + (mp[0] + mp[1]) * (1.0 / deg)[:, None]
    o = jnp.dot(h_ref[...], wa_ref[...], preferred_element_type=jnp.float32)
    o += jnp.dot(hn, wb_ref[...], preferred_element_type=jnp.float32)
    o_ref[...] = o + b_ref[...][None, :]


def _tc_layer1(h, agg1, mp, dp, waT, wbT, b, d_out):
    return pl.pallas_call(
        _tc_layer1_body,
        grid=(_GRID,),
        in_specs=[
            pl.BlockSpec((_BN, D), lambda i: (i, 0)),
            pl.BlockSpec((_BN, D), lambda i: (i, 0)),
            pl.BlockSpec((NC, _BN, D), lambda i: (0, i, 0)),
            pl.BlockSpec((_BN, NW), lambda i: (i, 0)),
            pl.BlockSpec((D, d_out), lambda i: (0, 0)),
            pl.BlockSpec((D, d_out), lambda i: (0, 0)),
            pl.BlockSpec((d_out,), lambda i: (0,)),
        ],
        out_specs=pl.BlockSpec((_BN, d_out), lambda i: (i, 0)),
        out_shape=jax.ShapeDtypeStruct((N, d_out), jnp.float32),
    )(h, agg1, mp, dp, waT, wbT, b)


@jax.jit
def kernel(x, edge_index, hist0, agg_hist0, hist1, agg_hist1, W0, b0, W1, b1):
    d_out = W1.shape[0]
    src3d = edge_index[0].reshape(NW, G, CHG, K)
    dst3d = edge_index[1].reshape(NW, G, CHG, K)
    zfeat = jnp.zeros((N, D), jnp.float32)
    w0aT = W0[:, :D].T
    w0bT = W0[:, D:].T
    w1aT = W1[:, :D].T
    w1bT = W1[:, D:].T

    hdelta0 = _tc_sub(x, hist0)
    segsum_deg, segsum = _sc_kernels()
    mp0, dp = segsum_deg(hdelta0, src3d, dst3d, zfeat)
    dp = dp.T
    h, hdelta1 = _tc_layer0(x, agg_hist0, mp0, dp, w0aT, w0bT, b0, hist1)
    (mp1,) = segsum(hdelta1, src3d, dst3d, zfeat)
    return _tc_layer1(h, agg_hist1, mp1, dp, w1aT, w1bT, b1, d_out)


# speedup vs baseline: 7.9817x; 1.2330x over previous
import functools

import jax
import jax.numpy as jnp
from jax import lax
from jax.experimental import pallas as pl
from jax.experimental.pallas import tpu as pltpu
from jax.experimental.pallas import tpu_sc as plsc

N = 10000
E = 320000
D = 128
NC = 2
NS = 16
NW = NC * NS
EPW = E // NW
K = 80
CH = EPW // K
G = 5
CHG = CH // G
RPS = 624
TAIL = N - NS * RPS
TBASE = NS * RPS


def _sc_segsum_body(do_deg, *refs):
    if do_deg:
        (hdelta, src3d, dst3d, zfeat, msum_out, deg_out,
         acc, src_idx, dst_idx, rows0, rows1, dloc, sem0, sem1) = refs
    else:
        (hdelta, src3d, dst3d, zfeat, msum_out,
         acc, src_idx, dst_idx, rows0, rows1, sem0, sem1) = refs

    c = lax.axis_index("c")
    s = lax.axis_index("s")
    w = c * NS + s

    base = s * RPS
    pltpu.sync_copy(zfeat.at[pl.ds(base, RPS)], acc.at[pl.ds(base, RPS)])

    @pl.when(s == NS - 1)
    def _zero_tail():
        pltpu.sync_copy(zfeat.at[pl.ds(TBASE, TAIL)],
                        acc.at[pl.ds(TBASE, TAIL)])

    if do_deg:
        def zero_deg(i, _):
            dloc[pl.ds(i * 16, 16)] = jnp.zeros((16,), jnp.float32)
            return 0
        lax.fori_loop(0, N // 16, zero_deg, 0)
    del base

    plsc.subcore_barrier()

    def deg_vec(j):
        if do_deg:
            def vec(l, _):
                iv = dst_idx[j, pl.ds(l * 16, 16)]
                plsc.addupdate_scatter(
                    dloc, [iv], jnp.ones((16,), jnp.float32))
                return 0
            lax.fori_loop(0, K // 16, vec, 0)

    def group(g, _):
        pltpu.sync_copy(src3d.at[w, g], src_idx)
        pltpu.sync_copy(dst3d.at[w, g], dst_idx)
        pltpu.async_copy(hdelta.at[src_idx.at[0]], rows0, sem0)

        def pair(p, _):
            a = 2 * p
            pltpu.make_async_copy(hdelta.at[pl.ds(0, K)], rows0, sem0).wait()
            pltpu.async_copy(hdelta.at[src_idx.at[a + 1]], rows1, sem1)
            deg_vec(a)
            pltpu.sync_copy(rows0, acc.at[dst_idx.at[a]], add=True)
            pltpu.make_async_copy(hdelta.at[pl.ds(0, K)], rows1, sem1).wait()
            pltpu.async_copy(hdelta.at[src_idx.at[a + 2]], rows0, sem0)
            deg_vec(a + 1)
            pltpu.sync_copy(rows1, acc.at[dst_idx.at[a + 1]], add=True)
            return 0

        lax.fori_loop(0, (CHG - 1) // 2, pair, 0)
        pltpu.make_async_copy(hdelta.at[pl.ds(0, K)], rows0, sem0).wait()
        deg_vec(CHG - 1)
        pltpu.sync_copy(rows0, acc.at[dst_idx.at[CHG - 1]], add=True)
        return 0

    lax.fori_loop(0, G, group, 0)
    plsc.subcore_barrier()

    pltpu.sync_copy(acc.at[pl.ds(s * RPS, RPS)],
                    msum_out.at[c, pl.ds(s * RPS, RPS)])
    if do_deg:
        pltpu.sync_copy(dloc, deg_out.at[w])

    @pl.when(s == NS - 1)
    def _pub_tail():
        pltpu.sync_copy(acc.at[pl.ds(TBASE, TAIL)],
                        msum_out.at[c, pl.ds(TBASE, TAIL)])


@functools.cache
def _sc_kernels():
    mesh = plsc.VectorSubcoreMesh(core_axis_name="c", subcore_axis_name="s")
    segsum_deg = pl.kernel(
        functools.partial(_sc_segsum_body, True),
        out_type=[
            jax.ShapeDtypeStruct((NC, N, D), jnp.float32),
            jax.ShapeDtypeStruct((NW, N), jnp.float32),
        ],
        mesh=mesh,
        compiler_params=pltpu.CompilerParams(needs_layout_passes=False),
        scratch_types=[
            pltpu.VMEM_SHARED((N, D), jnp.float32),
            pltpu.VMEM((CHG, K), jnp.int32),
            pltpu.VMEM((CHG, K), jnp.int32),
            pltpu.VMEM((K, D), jnp.float32),
            pltpu.VMEM((K, D), jnp.float32),
            pltpu.VMEM((N,), jnp.float32),
            pltpu.SemaphoreType.DMA,
            pltpu.SemaphoreType.DMA,
        ],
    )
    segsum = pl.kernel(
        functools.partial(_sc_segsum_body, False),
        out_type=[jax.ShapeDtypeStruct((NC, N, D), jnp.float32)],
        mesh=mesh,
        scratch_types=[
            pltpu.VMEM_SHARED((N, D), jnp.float32),
            pltpu.VMEM((CHG, K), jnp.int32),
            pltpu.VMEM((CHG, K), jnp.int32),
            pltpu.VMEM((K, D), jnp.float32),
            pltpu.VMEM((K, D), jnp.float32),
            pltpu.SemaphoreType.DMA,
            pltpu.SemaphoreType.DMA,
        ],
    )
    return segsum_deg, segsum


_BN = 400
_GRID = N // _BN


def _tc_sub_body(x_ref, h_ref, o_ref):
    o_ref[...] = x_ref[...] - h_ref[...]


def _tc_sub(x, h):
    return pl.pallas_call(
        _tc_sub_body,
        grid=(_GRID,),
        in_specs=[
            pl.BlockSpec((_BN, D), lambda i: (i, 0)),
            pl.BlockSpec((_BN, D), lambda i: (i, 0)),
        ],
        out_specs=pl.BlockSpec((_BN, D), lambda i: (i, 0)),
        out_shape=jax.ShapeDtypeStruct((N, D), jnp.float32),
    )(x, h)


def _tc_layer0_body(x_ref, agg_ref, mp_ref, dp_ref, wa_ref, wb_ref, b_ref,
                    hist1_ref, h_ref, hd1_ref):
    mp = mp_ref[...]
    dp = dp_ref[...]
    deg = jnp.maximum(jnp.sum(dp, axis=1), 1.0)
    hn = agg_ref[...] + (mp[0] + mp[1]) * (1.0 / deg)[:, None]
    h = jnp.dot(x_ref[...], wa_ref[...], preferred_element_type=jnp.float32)
    h += jnp.dot(hn, wb_ref[...], preferred_element_type=jnp.float32)
    h = jnp.maximum(h + b_ref[...][None, :], 0.0)
    h_ref[...] = h
    hd1_ref[...] = h - hist1_ref[...]


def _tc_layer0(x, agg0, mp, dp, waT, wbT, b, hist1):
    return pl.pallas_call(
        _tc_layer0_body,
        grid=(_GRID,),
        in_specs=[
            pl.BlockSpec((_BN, D), lambda i: (i, 0)),
            pl.BlockSpec((_BN, D), lambda i: (i, 0)),
            pl.BlockSpec((NC, _BN, D), lambda i: (0, i, 0)),
            pl.BlockSpec((_BN, NW), lambda i: (i, 0)),
            pl.BlockSpec((D, D), lambda i: (0, 0)),
            pl.BlockSpec((D, D), lambda i: (0, 0)),
            pl.BlockSpec((D,), lambda i: (0,)),
            pl.BlockSpec((_BN, D), lambda i: (i, 0)),
        ],
        out_specs=[
            pl.BlockSpec((_BN, D), lambda i: (i, 0)),
            pl.BlockSpec((_BN, D), lambda i: (i, 0)),
        ],
        out_shape=[
            jax.ShapeDtypeStruct((N, D), jnp.float32),
            jax.ShapeDtypeStruct((N, D), jnp.float32),
        ],
    )(x, agg0, mp, dp, waT, wbT, b, hist1)


def _tc_layer1_body(h_ref, agg_ref, mp_ref, dp_ref, wa_ref, wb_ref, b_ref,
                    o_ref):
    mp = mp_ref[...]
    dp = dp_ref[...]
    deg = jnp.maximum(jnp.sum(dp, axis=1), 1.0)
    hn = agg_ref[...] + (mp[0] + mp[1]) * (1.0 / deg)[:, None]
    o = jnp.dot(h_ref[...], wa_ref[...], preferred_element_type=jnp.float32)
    o += jnp.dot(hn, wb_ref[...], preferred_element_type=jnp.float32)
    o_ref[...] = o + b_ref[...][None, :]


def _tc_layer1(h, agg1, mp, dp, waT, wbT, b, d_out):
    return pl.pallas_call(
        _tc_layer1_body,
        grid=(_GRID,),
        in_specs=[
            pl.BlockSpec((_BN, D), lambda i: (i, 0)),
            pl.BlockSpec((_BN, D), lambda i: (i, 0)),
            pl.BlockSpec((NC, _BN, D), lambda i: (0, i, 0)),
            pl.BlockSpec((_BN, NW), lambda i: (i, 0)),
            pl.BlockSpec((D, d_out), lambda i: (0, 0)),
            pl.BlockSpec((D, d_out), lambda i: (0, 0)),
            pl.BlockSpec((d_out,), lambda i: (0,)),
        ],
        out_specs=pl.BlockSpec((_BN, d_out), lambda i: (i, 0)),
        out_shape=jax.ShapeDtypeStruct((N, d_out), jnp.float32),
    )(h, agg1, mp, dp, waT, wbT, b)


@jax.jit
def kernel(x, edge_index, hist0, agg_hist0, hist1, agg_hist1, W0, b0, W1, b1):
    d_out = W1.shape[0]
    src3d = edge_index[0].reshape(NW, G, CHG, K)
    dst3d = edge_index[1].reshape(NW, G, CHG, K)
    zfeat = jnp.zeros((N, D), jnp.float32)
    w0aT = W0[:, :D].T
    w0bT = W0[:, D:].T
    w1aT = W1[:, :D].T
    w1bT = W1[:, D:].T

    hdelta0 = _tc_sub(x, hist0)
    segsum_deg, segsum = _sc_kernels()
    mp0, dp = segsum_deg(hdelta0, src3d, dst3d, zfeat)
    dp = dp.T
    h, hdelta1 = _tc_layer0(x, agg_hist0, mp0, dp, w0aT, w0bT, b0, hist1)
    (mp1,) = segsum(hdelta1, src3d, dst3d, zfeat)
    return _tc_layer1(h, agg_hist1, mp1, dp, w1aT, w1bT, b1, d_out)
